# baseline (speedup 1.0000x reference)
import functools

import jax
import jax.numpy as jnp
from jax import lax
from jax.experimental import pallas as pl
from jax.experimental.pallas import tpu as pltpu
from jax.experimental.pallas import tpu_sc as plsc

N = 10000
E = 160000
D = 256
DH = D // 2
NC = 2
NS = 16
LANES = 16
DEGW = 128

EB = 96
NBT = 106
EPT = NBT * EB
E_PAD = NS * EPT

ACC_ROWS = 10080
NZBLK = ACC_ROWS // EB
CB = 80
OBLKS = N // CB

MBLK = 400
GRID = N // MBLK

_mesh = plsc.VectorSubcoreMesh(core_axis_name="c", subcore_axis_name="s")


def _fill_2d(ref, rows, width, value):
  def body(r, _):
    for j in range(width // LANES):
      ref[r, pl.ds(j * LANES, LANES)] = jnp.full((LANES,), value, jnp.float32)
    return 0
  lax.fori_loop(0, rows, body, 0)


def _zero_acc(acc_sh, zbuf, s):
  nblk = jnp.where(s < NZBLK % NS, NZBLK // NS + 1, NZBLK // NS)
  def body(k, _):
    row = (s + k * NS) * EB
    pltpu.sync_copy(zbuf, acc_sh.at[pl.ds(row, EB)])
    return 0
  lax.fori_loop(0, nblk, body, 0)


def _copy_out(acc_sh, out_hbm, s):
  nblk = jnp.where(s < OBLKS % NS, OBLKS // NS + 1, OBLKS // NS)
  def body(t, _):
    row = (s + t * NS) * CB
    pltpu.sync_copy(acc_sh.at[pl.ds(row, CB)], out_hbm.at[pl.ds(row, CB)])
    return 0
  lax.fori_loop(0, nblk, body, 0)


@functools.partial(
    pl.kernel,
    out_type=(jax.ShapeDtypeStruct((N, DEGW), jnp.float32),
              jax.ShapeDtypeStruct((N, DEGW), jnp.float32)),
    mesh=_mesh,
    scratch_types=[
        pltpu.VMEM((EB,), jnp.int32),
        pltpu.VMEM((EB,), jnp.int32),
        pltpu.VMEM((EB, DEGW), jnp.float32),
        pltpu.VMEM_SHARED((ACC_ROWS, DEGW), jnp.float32),
        pltpu.SemaphoreType.DMA,
        pltpu.SemaphoreType.DMA,
    ],
)
def _deg_kernel(src_hbm, dst_hbm, degout_hbm, degin_hbm,
                ib0, ib1, pay_v, acc_sh, is0, is1):
  c = lax.axis_index("c")
  s = lax.axis_index("s")

  _fill_2d(pay_v, EB, DEGW, 0.0)
  _zero_acc(acc_sh, pay_v, s)
  plsc.subcore_barrier()

  _fill_2d(pay_v, EB, DEGW, 1.0)

  def run(idx_hbm, out_hbm):
    def idx_slice(i):
      return idx_hbm.at[pl.ds(s * EPT + i * EB, EB)]
    pltpu.sync_copy(idx_slice(0), ib0)
    def body(t, _):
      i0 = 2 * t
      i1 = i0 + 1
      pltpu.async_copy(idx_slice(i1), ib1, is1)
      pltpu.sync_copy(pay_v, acc_sh.at[ib0], add=True)
      @pl.when(t < NBT // 2 - 1)
      def _():
        pltpu.async_copy(idx_slice(i0 + 2), ib0, is0)
      pltpu.make_async_copy(idx_slice(i1), ib1, is1).wait()
      pltpu.sync_copy(pay_v, acc_sh.at[ib1], add=True)
      @pl.when(t < NBT // 2 - 1)
      def _():
        pltpu.make_async_copy(idx_slice(i0 + 2), ib0, is0).wait()
      return 0
    lax.fori_loop(0, NBT // 2, body, 0)
    plsc.subcore_barrier()
    _copy_out(acc_sh, out_hbm, s)

  @pl.when(c == 0)
  def _():
    run(src_hbm, degout_hbm)

  @pl.when(c == 1)
  def _():
    run(dst_hbm, degin_hbm)


@functools.partial(
    pl.kernel,
    out_type=(jax.ShapeDtypeStruct((N, DH), jnp.float32),
              jax.ShapeDtypeStruct((N, DH), jnp.float32)),
    mesh=_mesh,
    scratch_types=[
        pltpu.VMEM((EB,), jnp.int32),
        pltpu.VMEM((EB,), jnp.int32),
        pltpu.VMEM((EB,), jnp.int32),
        pltpu.VMEM((EB,), jnp.int32),
        pltpu.VMEM((EB, DH), jnp.float32),
        pltpu.VMEM((EB, DH), jnp.float32),
        pltpu.VMEM_SHARED((ACC_ROWS, DH), jnp.float32),
        pltpu.SemaphoreType.DMA,
        pltpu.SemaphoreType.DMA,
    ],
)
def _agg_kernel(hl_hbm, hr_hbm, src_hbm, dst_hbm, outl_hbm, outr_hbm,
                sb0, db0, sb1, db1, m0, m1, acc_sh, g0, g1):
  c = lax.axis_index("c")
  s = lax.axis_index("s")

  _fill_2d(m0, EB, DH, 0.0)
  _zero_acc(acc_sh, m0, s)
  plsc.subcore_barrier()

  def run(h_hbm, out_hbm):
    def load_idx(i, sb, db):
      pltpu.sync_copy(src_hbm.at[pl.ds(s * EPT + i * EB, EB)], sb)
      pltpu.sync_copy(dst_hbm.at[pl.ds(s * EPT + i * EB, EB)], db)
    load_idx(0, sb0, db0)
    pltpu.async_copy(h_hbm.at[sb0], m0, g0)
    def body(t, _):
      i0 = 2 * t
      i1 = i0 + 1
      load_idx(i1, sb1, db1)
      pltpu.async_copy(h_hbm.at[sb1], m1, g1)
      pltpu.make_async_copy(h_hbm.at[sb0], m0, g0).wait()
      pltpu.sync_copy(m0, acc_sh.at[db0], add=True)
      @pl.when(t < NBT // 2 - 1)
      def _():
        load_idx(i0 + 2, sb0, db0)
        pltpu.async_copy(h_hbm.at[sb0], m0, g0)
      pltpu.make_async_copy(h_hbm.at[sb1], m1, g1).wait()
      pltpu.sync_copy(m1, acc_sh.at[db1], add=True)
      return 0
    lax.fori_loop(0, NBT // 2, body, 0)
    plsc.subcore_barrier()
    _copy_out(acc_sh, out_hbm, s)

  @pl.when(c == 0)
  def _():
    run(hl_hbm, outl_hbm)

  @pl.when(c == 1)
  def _():
    run(hr_hbm, outr_hbm)


def _prep_body(feat_ref, dow_ref, diw_ref, hl_ref, hr_ref, nsw_ref, ndw_ref):
  ns = lax.rsqrt(jnp.maximum(dow_ref[...], 1.0))
  nd = lax.rsqrt(jnp.maximum(diw_ref[...], 1.0))
  nsw_ref[...] = ns
  ndw_ref[...] = nd
  h0s = feat_ref[...] * ns[:, 0:1]
  hl_ref[...] = h0s[:, :DH]
  hr_ref[...] = h0s[:, DH:]


def _prep(features, degout_w, degin_w):
  return pl.pallas_call(
      _prep_body,
      grid=(GRID,),
      in_specs=[
          pl.BlockSpec((MBLK, D), lambda i: (i, 0)),
          pl.BlockSpec((MBLK, DEGW), lambda i: (i, 0)),
          pl.BlockSpec((MBLK, DEGW), lambda i: (i, 0)),
      ],
      out_specs=[
          pl.BlockSpec((MBLK, DH), lambda i: (i, 0)),
          pl.BlockSpec((MBLK, DH), lambda i: (i, 0)),
          pl.BlockSpec((MBLK, DEGW), lambda i: (i, 0)),
          pl.BlockSpec((MBLK, DEGW), lambda i: (i, 0)),
      ],
      out_shape=[
          jax.ShapeDtypeStruct((N, DH), jnp.float32),
          jax.ShapeDtypeStruct((N, DH), jnp.float32),
          jax.ShapeDtypeStruct((N, DEGW), jnp.float32),
          jax.ShapeDtypeStruct((N, DEGW), jnp.float32),
      ],
  )(features, degout_w, degin_w)


def _layer_body_mid(al_ref, ar_ref, ndw_ref, nsw_ref, w_ref, b_ref,
                    outl_ref, outr_ref):
  nd = ndw_ref[:, 0:1]
  y = (jnp.dot(al_ref[...] * nd, w_ref[:DH, :],
               preferred_element_type=jnp.float32)
       + jnp.dot(ar_ref[...] * nd, w_ref[DH:, :],
                 preferred_element_type=jnp.float32)
       + b_ref[0:1, :])
  y = jnp.maximum(y, 0.0) * nsw_ref[:, 0:1]
  outl_ref[...] = y[:, :DH]
  outr_ref[...] = y[:, DH:]


def _layer_body_last(al_ref, ar_ref, ndw_ref, nsw_ref, w_ref, b_ref, out_ref):
  nd = ndw_ref[:, 0:1]
  out_ref[...] = (jnp.dot(al_ref[...] * nd, w_ref[:DH, :],
                          preferred_element_type=jnp.float32)
                  + jnp.dot(ar_ref[...] * nd, w_ref[DH:, :],
                            preferred_element_type=jnp.float32)
                  + b_ref[0:1, :])


def _layer(al, ar, ndw, nsw, w, b, last):
  if last:
    out_specs = pl.BlockSpec((MBLK, D), lambda i: (i, 0))
    out_shape = jax.ShapeDtypeStruct((N, D), jnp.float32)
    body = _layer_body_last
  else:
    out_specs = [pl.BlockSpec((MBLK, DH), lambda i: (i, 0)),
                 pl.BlockSpec((MBLK, DH), lambda i: (i, 0))]
    out_shape = [jax.ShapeDtypeStruct((N, DH), jnp.float32),
                 jax.ShapeDtypeStruct((N, DH), jnp.float32)]
    body = _layer_body_mid
  return pl.pallas_call(
      body,
      grid=(GRID,),
      in_specs=[
          pl.BlockSpec((MBLK, DH), lambda i: (i, 0)),
          pl.BlockSpec((MBLK, DH), lambda i: (i, 0)),
          pl.BlockSpec((MBLK, DEGW), lambda i: (i, 0)),
          pl.BlockSpec((MBLK, DEGW), lambda i: (i, 0)),
          pl.BlockSpec((D, D), lambda i: (0, 0)),
          pl.BlockSpec((1, D), lambda i: (0, 0)),
      ],
      out_specs=out_specs,
      out_shape=out_shape,
  )(al, ar, ndw, nsw, w, b.reshape(1, D))


def kernel(features, edge_index, W1, b1, W2, b2, W3, b3):
  src = edge_index[0].astype(jnp.int32)
  dst = edge_index[1].astype(jnp.int32)

  pad_valid = jnp.zeros((E_PAD - E,), jnp.int32)
  pad_dummy = jnp.full((E_PAD - E,), N, jnp.int32)
  src_agg = jnp.concatenate([src, pad_valid])
  src_deg = jnp.concatenate([src, pad_dummy])
  dst_pad = jnp.concatenate([dst, pad_dummy])

  degout_w, degin_w = _deg_kernel(src_deg, dst_pad)
  hl, hr, nsw, ndw = _prep(features, degout_w, degin_w)

  al, ar = _agg_kernel(hl, hr, src_agg, dst_pad)
  hl, hr = _layer(al, ar, ndw, nsw, W1, b1, last=False)

  al, ar = _agg_kernel(hl, hr, src_agg, dst_pad)
  hl, hr = _layer(al, ar, ndw, nsw, W2, b2, last=False)

  al, ar = _agg_kernel(hl, hr, src_agg, dst_pad)
  return _layer(al, ar, ndw, nsw, W3, b3, last=True)


# speedup vs baseline: 1.2293x; 1.2293x over previous
import functools

import jax
import jax.numpy as jnp
from jax import lax
from jax.experimental import pallas as pl
from jax.experimental.pallas import tpu as pltpu
from jax.experimental.pallas import tpu_sc as plsc

N = 10000
E = 160000
D = 256
DH = D // 2
NC = 2
NS = 16
LANES = 16
DEGW = 128

EB = 80
NBT = 126
EPT = NBT * EB
E_PAD = NS * EPT

ACC_ROWS = 10080
NZBLK = ACC_ROWS // EB
OBLKS = N // EB

MBLK = 400
GRID = N // MBLK

_mesh = plsc.VectorSubcoreMesh(core_axis_name="c", subcore_axis_name="s")


def _fill_2d(ref, rows, width, value):
  def body(r, _):
    for j in range(width // LANES):
      ref[r, pl.ds(j * LANES, LANES)] = jnp.full((LANES,), value, jnp.float32)
    return 0
  lax.fori_loop(0, rows, body, 0)


def _zero_acc(acc_sh, zbuf, s):
  nblk = jnp.where(s < NZBLK % NS, NZBLK // NS + 1, NZBLK // NS)
  def body(k, _):
    row = (s + k * NS) * EB
    pltpu.sync_copy(zbuf, acc_sh.at[pl.ds(row, EB)])
    return 0
  lax.fori_loop(0, nblk, body, 0)


def _copy_out(acc_sh, out_hbm, s):
  nblk = jnp.where(s < OBLKS % NS, OBLKS // NS + 1, OBLKS // NS)
  def body(t, _):
    row = (s + t * NS) * EB
    pltpu.sync_copy(acc_sh.at[pl.ds(row, EB)], out_hbm.at[pl.ds(row, EB)])
    return 0
  lax.fori_loop(0, nblk, body, 0)


@functools.partial(
    pl.kernel,
    out_type=(jax.ShapeDtypeStruct((N, DEGW), jnp.float32),
              jax.ShapeDtypeStruct((N, DEGW), jnp.float32)),
    mesh=_mesh,
    scratch_types=[
        pltpu.VMEM((EB,), jnp.int32),
        pltpu.VMEM((EB,), jnp.int32),
        pltpu.VMEM((EB, DEGW), jnp.float32),
        pltpu.VMEM_SHARED((ACC_ROWS, DEGW), jnp.float32),
        pltpu.SemaphoreType.DMA,
        pltpu.SemaphoreType.DMA,
    ],
)
def _deg_kernel(src_hbm, dst_hbm, degout_hbm, degin_hbm,
                ib0, ib1, pay_v, acc_sh, is0, is1):
  c = lax.axis_index("c")
  s = lax.axis_index("s")

  _fill_2d(pay_v, EB, DEGW, 0.0)
  _zero_acc(acc_sh, pay_v, s)
  plsc.subcore_barrier()

  _fill_2d(pay_v, EB, DEGW, 1.0)

  def run(idx_hbm, out_hbm):
    def idx_slice(i):
      return idx_hbm.at[pl.ds(s * EPT + i * EB, EB)]
    pltpu.sync_copy(idx_slice(0), ib0)
    def body(t, _):
      i0 = 2 * t
      i1 = i0 + 1
      pltpu.async_copy(idx_slice(i1), ib1, is1)
      pltpu.sync_copy(pay_v, acc_sh.at[ib0], add=True)
      @pl.when(t < NBT // 2 - 1)
      def _():
        pltpu.async_copy(idx_slice(i0 + 2), ib0, is0)
      pltpu.make_async_copy(idx_slice(i1), ib1, is1).wait()
      pltpu.sync_copy(pay_v, acc_sh.at[ib1], add=True)
      @pl.when(t < NBT // 2 - 1)
      def _():
        pltpu.make_async_copy(idx_slice(i0 + 2), ib0, is0).wait()
      return 0
    lax.fori_loop(0, NBT // 2, body, 0)
    plsc.subcore_barrier()
    _copy_out(acc_sh, out_hbm, s)

  @pl.when(c == 0)
  def _():
    run(src_hbm, degout_hbm)

  @pl.when(c == 1)
  def _():
    run(dst_hbm, degin_hbm)


@functools.partial(
    pl.kernel,
    out_type=(jax.ShapeDtypeStruct((N, DH), jnp.float32),
              jax.ShapeDtypeStruct((N, DH), jnp.float32)),
    mesh=_mesh,
    scratch_types=[
        pltpu.VMEM((EB,), jnp.int32),
        pltpu.VMEM((EB,), jnp.int32),
        pltpu.VMEM((EB,), jnp.int32),
        pltpu.VMEM((EB,), jnp.int32),
        pltpu.VMEM((EB, DH), jnp.float32),
        pltpu.VMEM((EB, DH), jnp.float32),
        pltpu.VMEM_SHARED((ACC_ROWS, DH), jnp.float32),
        pltpu.SemaphoreType.DMA,
        pltpu.SemaphoreType.DMA,
    ],
)
def _agg_kernel(hl_hbm, hr_hbm, src_hbm, dst_hbm, outl_hbm, outr_hbm,
                sb0, db0, sb1, db1, m0, m1, acc_sh, g0, g1):
  c = lax.axis_index("c")
  s = lax.axis_index("s")

  _fill_2d(m0, EB, DH, 0.0)
  _zero_acc(acc_sh, m0, s)
  plsc.subcore_barrier()

  def run(h_hbm, out_hbm):
    def load_idx(i, sb, db):
      pltpu.sync_copy(src_hbm.at[pl.ds(s * EPT + i * EB, EB)], sb)
      pltpu.sync_copy(dst_hbm.at[pl.ds(s * EPT + i * EB, EB)], db)
    load_idx(0, sb0, db0)
    pltpu.async_copy(h_hbm.at[sb0], m0, g0)
    def body(t, _):
      i0 = 2 * t
      i1 = i0 + 1
      load_idx(i1, sb1, db1)
      pltpu.async_copy(h_hbm.at[sb1], m1, g1)
      pltpu.make_async_copy(h_hbm.at[sb0], m0, g0).wait()
      pltpu.sync_copy(m0, acc_sh.at[db0], add=True)
      @pl.when(t < NBT // 2 - 1)
      def _():
        load_idx(i0 + 2, sb0, db0)
        pltpu.async_copy(h_hbm.at[sb0], m0, g0)
      pltpu.make_async_copy(h_hbm.at[sb1], m1, g1).wait()
      pltpu.sync_copy(m1, acc_sh.at[db1], add=True)
      return 0
    lax.fori_loop(0, NBT // 2, body, 0)
    plsc.subcore_barrier()
    _copy_out(acc_sh, out_hbm, s)

  @pl.when(c == 0)
  def _():
    run(hl_hbm, outl_hbm)

  @pl.when(c == 1)
  def _():
    run(hr_hbm, outr_hbm)


def _prep_body(feat_ref, dow_ref, diw_ref, hl_ref, hr_ref, nsw_ref, ndw_ref):
  ns = lax.rsqrt(jnp.maximum(dow_ref[...], 1.0))
  nd = lax.rsqrt(jnp.maximum(diw_ref[...], 1.0))
  nsw_ref[...] = ns
  ndw_ref[...] = nd
  h0s = feat_ref[...] * ns[:, 0:1]
  hl_ref[...] = h0s[:, :DH]
  hr_ref[...] = h0s[:, DH:]


def _prep(features, degout_w, degin_w):
  return pl.pallas_call(
      _prep_body,
      grid=(GRID,),
      in_specs=[
          pl.BlockSpec((MBLK, D), lambda i: (i, 0)),
          pl.BlockSpec((MBLK, DEGW), lambda i: (i, 0)),
          pl.BlockSpec((MBLK, DEGW), lambda i: (i, 0)),
      ],
      out_specs=[
          pl.BlockSpec((MBLK, DH), lambda i: (i, 0)),
          pl.BlockSpec((MBLK, DH), lambda i: (i, 0)),
          pl.BlockSpec((MBLK, DEGW), lambda i: (i, 0)),
          pl.BlockSpec((MBLK, DEGW), lambda i: (i, 0)),
      ],
      out_shape=[
          jax.ShapeDtypeStruct((N, DH), jnp.float32),
          jax.ShapeDtypeStruct((N, DH), jnp.float32),
          jax.ShapeDtypeStruct((N, DEGW), jnp.float32),
          jax.ShapeDtypeStruct((N, DEGW), jnp.float32),
      ],
  )(features, degout_w, degin_w)


def _layer_body_mid(al_ref, ar_ref, ndw_ref, nsw_ref, w_ref, b_ref,
                    outl_ref, outr_ref):
  nd = ndw_ref[:, 0:1]
  y = (jnp.dot(al_ref[...] * nd, w_ref[:DH, :],
               preferred_element_type=jnp.float32)
       + jnp.dot(ar_ref[...] * nd, w_ref[DH:, :],
                 preferred_element_type=jnp.float32)
       + b_ref[0:1, :])
  y = jnp.maximum(y, 0.0) * nsw_ref[:, 0:1]
  outl_ref[...] = y[:, :DH]
  outr_ref[...] = y[:, DH:]


def _layer_body_last(al_ref, ar_ref, ndw_ref, nsw_ref, w_ref, b_ref, out_ref):
  nd = ndw_ref[:, 0:1]
  out_ref[...] = (jnp.dot(al_ref[...] * nd, w_ref[:DH, :],
                          preferred_element_type=jnp.float32)
                  + jnp.dot(ar_ref[...] * nd, w_ref[DH:, :],
                            preferred_element_type=jnp.float32)
                  + b_ref[0:1, :])


def _layer(al, ar, ndw, nsw, w, b, last):
  if last:
    out_specs = pl.BlockSpec((MBLK, D), lambda i: (i, 0))
    out_shape = jax.ShapeDtypeStruct((N, D), jnp.float32)
    body = _layer_body_last
  else:
    out_specs = [pl.BlockSpec((MBLK, DH), lambda i: (i, 0)),
                 pl.BlockSpec((MBLK, DH), lambda i: (i, 0))]
    out_shape = [jax.ShapeDtypeStruct((N, DH), jnp.float32),
                 jax.ShapeDtypeStruct((N, DH), jnp.float32)]
    body = _layer_body_mid
  return pl.pallas_call(
      body,
      grid=(GRID,),
      in_specs=[
          pl.BlockSpec((MBLK, DH), lambda i: (i, 0)),
          pl.BlockSpec((MBLK, DH), lambda i: (i, 0)),
          pl.BlockSpec((MBLK, DEGW), lambda i: (i, 0)),
          pl.BlockSpec((MBLK, DEGW), lambda i: (i, 0)),
          pl.BlockSpec((D, D), lambda i: (0, 0)),
          pl.BlockSpec((1, D), lambda i: (0, 0)),
      ],
      out_specs=out_specs,
      out_shape=out_shape,
  )(al, ar, ndw, nsw, w, b.reshape(1, D))


def kernel(features, edge_index, W1, b1, W2, b2, W3, b3):
  src = edge_index[0].astype(jnp.int32)
  dst = edge_index[1].astype(jnp.int32)

  pad_valid = jnp.zeros((E_PAD - E,), jnp.int32)
  pad_dummy = jnp.full((E_PAD - E,), N, jnp.int32)
  src_agg = jnp.concatenate([src, pad_valid])
  src_deg = jnp.concatenate([src, pad_dummy])
  dst_pad = jnp.concatenate([dst, pad_dummy])

  degout_w, degin_w = _deg_kernel(src_deg, dst_pad)
  hl, hr, nsw, ndw = _prep(features, degout_w, degin_w)

  al, ar = _agg_kernel(hl, hr, src_agg, dst_pad)
  hl, hr = _layer(al, ar, ndw, nsw, W1, b1, last=False)

  al, ar = _agg_kernel(hl, hr, src_agg, dst_pad)
  hl, hr = _layer(al, ar, ndw, nsw, W2, b2, last=False)

  al, ar = _agg_kernel(hl, hr, src_agg, dst_pad)
  return _layer(al, ar, ndw, nsw, W3, b3, last=True)


# speedup vs baseline: 1.2329x; 1.0030x over previous
import functools

import jax
import jax.numpy as jnp
from jax import lax
from jax.experimental import pallas as pl
from jax.experimental.pallas import tpu as pltpu
from jax.experimental.pallas import tpu_sc as plsc

N = 10000
E = 160000
D = 256
DH = D // 2
NC = 2
NS = 16
LANES = 16
DEGW = 128

EB = 80
NBT = 126
EPT = NBT * EB
E_PAD = NS * EPT

ACC_ROWS = 10080
NZBLK = ACC_ROWS // EB
OBLKS = N // EB

MBLK = 400
GRID = N // MBLK

_mesh = plsc.VectorSubcoreMesh(core_axis_name="c", subcore_axis_name="s")


def _fill_2d(ref, rows, width, value):
  def body(r, _):
    for j in range(width // LANES):
      ref[r, pl.ds(j * LANES, LANES)] = jnp.full((LANES,), value, jnp.float32)
    return 0
  lax.fori_loop(0, rows, body, 0)


def _zero_acc(acc_sh, zbuf, s):
  nblk = jnp.where(s < NZBLK % NS, NZBLK // NS + 1, NZBLK // NS)
  def body(k, _):
    row = (s + k * NS) * EB
    pltpu.sync_copy(zbuf, acc_sh.at[pl.ds(row, EB)])
    return 0
  lax.fori_loop(0, nblk, body, 0)


def _copy_out(acc_sh, out_hbm, s):
  nblk = jnp.where(s < OBLKS % NS, OBLKS // NS + 1, OBLKS // NS)
  def body(t, _):
    row = (s + t * NS) * EB
    pltpu.sync_copy(acc_sh.at[pl.ds(row, EB)], out_hbm.at[pl.ds(row, EB)])
    return 0
  lax.fori_loop(0, nblk, body, 0)


@functools.partial(
    pl.kernel,
    out_type=(jax.ShapeDtypeStruct((N, DEGW), jnp.float32),
              jax.ShapeDtypeStruct((N, DEGW), jnp.float32)),
    mesh=_mesh,
    scratch_types=[
        pltpu.VMEM((EB,), jnp.int32),
        pltpu.VMEM((EB,), jnp.int32),
        pltpu.VMEM((EB, DEGW), jnp.float32),
        pltpu.VMEM_SHARED((ACC_ROWS, DEGW), jnp.float32),
        pltpu.SemaphoreType.DMA,
        pltpu.SemaphoreType.DMA,
    ],
)
def _deg_kernel(src_hbm, dst_hbm, degout_hbm, degin_hbm,
                ib0, ib1, pay_v, acc_sh, is0, is1):
  c = lax.axis_index("c")
  s = lax.axis_index("s")

  _fill_2d(pay_v, EB, DEGW, 0.0)
  _zero_acc(acc_sh, pay_v, s)
  plsc.subcore_barrier()

  _fill_2d(pay_v, EB, DEGW, 1.0)

  def run(idx_hbm, out_hbm):
    def idx_slice(i):
      return idx_hbm.at[pl.ds(s * EPT + i * EB, EB)]
    pltpu.sync_copy(idx_slice(0), ib0)
    def body(t, _):
      i0 = 2 * t
      i1 = i0 + 1
      pltpu.async_copy(idx_slice(i1), ib1, is1)
      pltpu.sync_copy(pay_v, acc_sh.at[ib0], add=True)
      @pl.when(t < NBT // 2 - 1)
      def _():
        pltpu.async_copy(idx_slice(i0 + 2), ib0, is0)
      pltpu.make_async_copy(idx_slice(i1), ib1, is1).wait()
      pltpu.sync_copy(pay_v, acc_sh.at[ib1], add=True)
      @pl.when(t < NBT // 2 - 1)
      def _():
        pltpu.make_async_copy(idx_slice(i0 + 2), ib0, is0).wait()
      return 0
    lax.fori_loop(0, NBT // 2, body, 0)
    plsc.subcore_barrier()
    _copy_out(acc_sh, out_hbm, s)

  @pl.when(c == 0)
  def _():
    run(src_hbm, degout_hbm)

  @pl.when(c == 1)
  def _():
    run(dst_hbm, degin_hbm)


@functools.partial(
    pl.kernel,
    out_type=(jax.ShapeDtypeStruct((N, DH), jnp.float32),
              jax.ShapeDtypeStruct((N, DH), jnp.float32)),
    mesh=_mesh,
    scratch_types=[
        pltpu.VMEM((EB,), jnp.int32),
        pltpu.VMEM((EB,), jnp.int32),
        pltpu.VMEM((EB,), jnp.int32),
        pltpu.VMEM((EB,), jnp.int32),
        pltpu.VMEM((EB,), jnp.int32),
        pltpu.VMEM((EB,), jnp.int32),
        pltpu.VMEM((EB, DH), jnp.float32),
        pltpu.VMEM((EB, DH), jnp.float32),
        pltpu.VMEM((EB, DH), jnp.float32),
        pltpu.VMEM_SHARED((ACC_ROWS, DH), jnp.float32),
        pltpu.SemaphoreType.DMA,
        pltpu.SemaphoreType.DMA,
        pltpu.SemaphoreType.DMA,
    ],
)
def _agg_kernel(hl_hbm, hr_hbm, src_hbm, dst_hbm, outl_hbm, outr_hbm,
                sb0, db0, sb1, db1, sb2, db2, m0, m1, m2, acc_sh, g0, g1, g2):
  c = lax.axis_index("c")
  s = lax.axis_index("s")

  _fill_2d(m0, EB, DH, 0.0)
  _zero_acc(acc_sh, m0, s)
  plsc.subcore_barrier()

  def run(h_hbm, out_hbm):
    def load_idx(i, sb, db):
      pltpu.sync_copy(src_hbm.at[pl.ds(s * EPT + i * EB, EB)], sb)
      pltpu.sync_copy(dst_hbm.at[pl.ds(s * EPT + i * EB, EB)], db)
    load_idx(0, sb0, db0)
    pltpu.async_copy(h_hbm.at[sb0], m0, g0)
    load_idx(1, sb1, db1)
    pltpu.async_copy(h_hbm.at[sb1], m1, g1)
    load_idx(2, sb2, db2)
    pltpu.async_copy(h_hbm.at[sb2], m2, g2)
    def step(t, i, sb, db, m, g):
      pltpu.make_async_copy(h_hbm.at[sb], m, g).wait()
      pltpu.sync_copy(m, acc_sh.at[db], add=True)
      @pl.when(t < NBT // 3 - 1)
      def _():
        load_idx(i + 3, sb, db)
        pltpu.async_copy(h_hbm.at[sb], m, g)
    def body(t, _):
      i0 = 3 * t
      step(t, i0, sb0, db0, m0, g0)
      step(t, i0 + 1, sb1, db1, m1, g1)
      step(t, i0 + 2, sb2, db2, m2, g2)
      return 0
    lax.fori_loop(0, NBT // 3, body, 0)
    plsc.subcore_barrier()
    _copy_out(acc_sh, out_hbm, s)

  @pl.when(c == 0)
  def _():
    run(hl_hbm, outl_hbm)

  @pl.when(c == 1)
  def _():
    run(hr_hbm, outr_hbm)


def _prep_body(feat_ref, dow_ref, diw_ref, hl_ref, hr_ref, nsw_ref, ndw_ref):
  ns = lax.rsqrt(jnp.maximum(dow_ref[...], 1.0))
  nd = lax.rsqrt(jnp.maximum(diw_ref[...], 1.0))
  nsw_ref[...] = ns
  ndw_ref[...] = nd
  h0s = feat_ref[...] * ns[:, 0:1]
  hl_ref[...] = h0s[:, :DH]
  hr_ref[...] = h0s[:, DH:]


def _prep(features, degout_w, degin_w):
  return pl.pallas_call(
      _prep_body,
      grid=(GRID,),
      in_specs=[
          pl.BlockSpec((MBLK, D), lambda i: (i, 0)),
          pl.BlockSpec((MBLK, DEGW), lambda i: (i, 0)),
          pl.BlockSpec((MBLK, DEGW), lambda i: (i, 0)),
      ],
      out_specs=[
          pl.BlockSpec((MBLK, DH), lambda i: (i, 0)),
          pl.BlockSpec((MBLK, DH), lambda i: (i, 0)),
          pl.BlockSpec((MBLK, DEGW), lambda i: (i, 0)),
          pl.BlockSpec((MBLK, DEGW), lambda i: (i, 0)),
      ],
      out_shape=[
          jax.ShapeDtypeStruct((N, DH), jnp.float32),
          jax.ShapeDtypeStruct((N, DH), jnp.float32),
          jax.ShapeDtypeStruct((N, DEGW), jnp.float32),
          jax.ShapeDtypeStruct((N, DEGW), jnp.float32),
      ],
  )(features, degout_w, degin_w)


def _layer_body_mid(al_ref, ar_ref, ndw_ref, nsw_ref, w_ref, b_ref,
                    outl_ref, outr_ref):
  nd = ndw_ref[:, 0:1]
  y = (jnp.dot(al_ref[...] * nd, w_ref[:DH, :],
               preferred_element_type=jnp.float32)
       + jnp.dot(ar_ref[...] * nd, w_ref[DH:, :],
                 preferred_element_type=jnp.float32)
       + b_ref[0:1, :])
  y = jnp.maximum(y, 0.0) * nsw_ref[:, 0:1]
  outl_ref[...] = y[:, :DH]
  outr_ref[...] = y[:, DH:]


def _layer_body_last(al_ref, ar_ref, ndw_ref, nsw_ref, w_ref, b_ref, out_ref):
  nd = ndw_ref[:, 0:1]
  out_ref[...] = (jnp.dot(al_ref[...] * nd, w_ref[:DH, :],
                          preferred_element_type=jnp.float32)
                  + jnp.dot(ar_ref[...] * nd, w_ref[DH:, :],
                            preferred_element_type=jnp.float32)
                  + b_ref[0:1, :])


def _layer(al, ar, ndw, nsw, w, b, last):
  if last:
    out_specs = pl.BlockSpec((MBLK, D), lambda i: (i, 0))
    out_shape = jax.ShapeDtypeStruct((N, D), jnp.float32)
    body = _layer_body_last
  else:
    out_specs = [pl.BlockSpec((MBLK, DH), lambda i: (i, 0)),
                 pl.BlockSpec((MBLK, DH), lambda i: (i, 0))]
    out_shape = [jax.ShapeDtypeStruct((N, DH), jnp.float32),
                 jax.ShapeDtypeStruct((N, DH), jnp.float32)]
    body = _layer_body_mid
  return pl.pallas_call(
      body,
      grid=(GRID,),
      in_specs=[
          pl.BlockSpec((MBLK, DH), lambda i: (i, 0)),
          pl.BlockSpec((MBLK, DH), lambda i: (i, 0)),
          pl.BlockSpec((MBLK, DEGW), lambda i: (i, 0)),
          pl.BlockSpec((MBLK, DEGW), lambda i: (i, 0)),
          pl.BlockSpec((D, D), lambda i: (0, 0)),
          pl.BlockSpec((1, D), lambda i: (0, 0)),
      ],
      out_specs=out_specs,
      out_shape=out_shape,
  )(al, ar, ndw, nsw, w, b.reshape(1, D))


def kernel(features, edge_index, W1, b1, W2, b2, W3, b3):
  src = edge_index[0].astype(jnp.int32)
  dst = edge_index[1].astype(jnp.int32)

  pad_valid = jnp.zeros((E_PAD - E,), jnp.int32)
  pad_dummy = jnp.full((E_PAD - E,), N, jnp.int32)
  src_agg = jnp.concatenate([src, pad_valid])
  src_deg = jnp.concatenate([src, pad_dummy])
  dst_pad = jnp.concatenate([dst, pad_dummy])

  degout_w, degin_w = _deg_kernel(src_deg, dst_pad)
  hl, hr, nsw, ndw = _prep(features, degout_w, degin_w)

  al, ar = _agg_kernel(hl, hr, src_agg, dst_pad)
  hl, hr = _layer(al, ar, ndw, nsw, W1, b1, last=False)

  al, ar = _agg_kernel(hl, hr, src_agg, dst_pad)
  hl, hr = _layer(al, ar, ndw, nsw, W2, b2, last=False)

  al, ar = _agg_kernel(hl, hr, src_agg, dst_pad)
  return _layer(al, ar, ndw, nsw, W3, b3, last=True)
